# baseline (speedup 1.0000x reference)
import functools

import jax
import jax.numpy as jnp
from jax.experimental import pallas as pl
from jax.experimental.pallas import tpu as pltpu
from jax.experimental.pallas import tpu_sc as plsc

VOCAB = 96
ECHAR = 50
EWORD = 64
KSIZE = 5
CPAD = 64
ROWW = 2 * CPAD


def _sc_gather(table2, idx2):
    info = plsc.get_sparse_core_info()
    nw = info.num_cores * info.num_subcores
    b = idx2.shape[0]
    b_per_w = b // nw
    ch = 128
    nsteps = b_per_w // ch
    mesh = plsc.VectorSubcoreMesh(core_axis_name="c", subcore_axis_name="s")

    @functools.partial(
        pl.kernel, mesh=mesh,
        out_type=jax.ShapeDtypeStruct((b, ROWW), jnp.float32),
        scratch_types=[
            pltpu.VMEM((ch,), jnp.int32),
            pltpu.VMEM((ch, ROWW), jnp.float32),
            pltpu.SemaphoreType.DMA,
        ],
    )
    def gather_kernel(table_hbm, idx_hbm, out_hbm, idx_v, rows_v, sem):
        wid = jax.lax.axis_index("s") * info.num_cores + jax.lax.axis_index("c")
        base = wid * b_per_w

        @pl.loop(0, nsteps)
        def _(i):
            off = base + i * ch
            pltpu.sync_copy(idx_hbm.at[pl.ds(off, ch)], idx_v)
            pltpu.async_copy(table_hbm.at[idx_v], rows_v, sem).wait()
            pltpu.sync_copy(rows_v, out_hbm.at[pl.ds(off, ch)])

    return gather_kernel(table2, idx2)


def _conv_body(nb, mw):
    npos = mw - KSIZE + 1
    nquads = npos // 4

    def body(e_ref, wflat_ref, wpg_ref, cb1_ref, bpg_ref,
             out_ref, wpad_ref, uquad_ref):
        @pl.when(pl.program_id(0) == 0)
        def _build_tables():
            zpad = jnp.zeros((CPAD - ECHAR, EWORD), jnp.float32)
            pieces = []
            for k in range(KSIZE):
                pieces.append(wflat_ref[k * ECHAR:(k + 1) * ECHAR, :])
                pieces.append(zpad)
            wpad = jnp.concatenate(pieces, axis=0)
            wpad_ref[...] = wpad.astype(jnp.bfloat16)
            z64 = jnp.zeros((CPAD, EWORD), jnp.float32)
            cols = []
            for p in range(4):
                cols.append(jnp.concatenate(
                    [z64] * p + [wpad] + [z64] * (3 - p), axis=0))
            uquad_ref[...] = jnp.concatenate(cols, axis=1).astype(jnp.bfloat16)

        e = e_ref[...].astype(jnp.bfloat16)
        uquad = uquad_ref[...]
        m256 = None
        for q in range(nquads):
            base = 4 * q * CPAD
            a = jax.lax.dot_general(
                e[:, base:base + 8 * CPAD], uquad,
                (((1,), (0,)), ((), ())),
                preferred_element_type=jnp.float32)
            m256 = a if m256 is None else jnp.maximum(m256, a)
        m = jnp.maximum(
            jnp.maximum(m256[:, :EWORD], m256[:, EWORD:2 * EWORD]),
            jnp.maximum(m256[:, 2 * EWORD:3 * EWORD], m256[:, 3 * EWORD:]))
        for t in range(4 * nquads, npos):
            base = t * CPAD
            a = jax.lax.dot_general(
                e[:, base:base + KSIZE * CPAD], wpad_ref[...],
                (((1,), (0,)), ((), ())),
                preferred_element_type=jnp.float32)
            m = jnp.maximum(m, a)
        m = jnp.maximum(m + cb1_ref[...], 0.0)

        h = jax.lax.dot_general(
            m.astype(jnp.bfloat16), wpg_ref[...], (((1,), (0,)), ((), ())),
            preferred_element_type=jnp.float32) + bpg_ref[...]
        proj = jnp.maximum(h[:, :EWORD], 0.0)
        gate = jax.nn.sigmoid(h[:, EWORD:])
        out_ref[...] = gate * proj + (1.0 - gate) * m

    return body


def _conv_call(e, wflat, wpg, cb1, bpg, n, mw, ncols):
    nb = 2048 if n % 2048 == 0 else n
    grid = (n // nb,)
    return pl.pallas_call(
        _conv_body(nb, mw),
        grid=grid,
        in_specs=[
            pl.BlockSpec((nb, ncols), lambda i: (i, 0)),
            pl.BlockSpec((KSIZE * ECHAR, EWORD), lambda i: (0, 0)),
            pl.BlockSpec((EWORD, 2 * EWORD), lambda i: (0, 0)),
            pl.BlockSpec((1, EWORD), lambda i: (0, 0)),
            pl.BlockSpec((1, 2 * EWORD), lambda i: (0, 0)),
        ],
        out_specs=pl.BlockSpec((nb, EWORD), lambda i: (i, 0)),
        out_shape=jax.ShapeDtypeStruct((n, EWORD), jnp.float32),
        scratch_shapes=[
            pltpu.VMEM((KSIZE * CPAD, EWORD), jnp.bfloat16),
            pltpu.VMEM((8 * CPAD, 4 * EWORD), jnp.bfloat16),
        ],
    )(e, wflat, wpg, cb1, bpg)


def kernel(input, emb_table, conv_w, conv_b, W_proj, b_proj, W_gate, b_gate):
    sl, bs, mw = input.shape
    n = sl * bs
    npair = (mw + 1) // 2

    embp = jnp.pad(emb_table, ((0, 0), (0, CPAD - ECHAR)))
    table2 = jnp.concatenate(
        [jnp.repeat(embp, VOCAB, axis=0), jnp.tile(embp, (VOCAB, 1))],
        axis=1)
    wflat = conv_w.transpose(2, 1, 0).reshape(KSIZE * ECHAR, EWORD)
    wpg = jnp.concatenate([W_proj.T, W_gate.T], axis=1).astype(jnp.bfloat16)
    cb1 = conv_b[None, :]
    bpg = jnp.concatenate([b_proj, b_gate])[None, :]

    ii = input.reshape(n, mw)
    iipad = jnp.concatenate(
        [ii, jnp.zeros((n, 2 * npair - mw), jnp.int32)], axis=1)
    pairs = iipad.reshape(n, npair, 2)
    idx2 = (pairs[..., 0] * VOCAB + pairs[..., 1]).reshape(n * npair)

    eg = _sc_gather(table2, idx2)
    e = eg.reshape(n, npair * ROWW)

    out = _conv_call(e, wflat, wpg, cb1, bpg, n, mw, npair * ROWW)
    return out.reshape(sl, bs, EWORD)


# speedup vs baseline: 1.1764x; 1.1764x over previous
import functools

import jax
import jax.numpy as jnp
from jax.experimental import pallas as pl
from jax.experimental.pallas import tpu as pltpu
from jax.experimental.pallas import tpu_sc as plsc

VOCAB = 96
ECHAR = 50
EWORD = 64
KSIZE = 5
CPAD = 64
ROWW = 2 * CPAD


def _sc_gather(table2, idx2):
    info = plsc.get_sparse_core_info()
    nw = info.num_cores * info.num_subcores
    b = idx2.shape[0]
    b_per_w = b // nw
    ch = 256
    nsteps = b_per_w // ch
    mesh = plsc.VectorSubcoreMesh(core_axis_name="c", subcore_axis_name="s")

    @functools.partial(
        pl.kernel, mesh=mesh,
        out_type=jax.ShapeDtypeStruct((b, ROWW), jnp.float32),
        scratch_types=[
            pltpu.VMEM((b_per_w,), jnp.int32),
            pltpu.VMEM((ch, ROWW), jnp.float32),
            pltpu.VMEM((ch, ROWW), jnp.float32),
            pltpu.SemaphoreType.DMA,
            pltpu.SemaphoreType.DMA,
            pltpu.SemaphoreType.DMA,
            pltpu.SemaphoreType.DMA,
        ],
    )
    def gather_kernel(table_hbm, idx_hbm, out_hbm, idx_v, buf0, buf1,
                      sg0, sg1, sw0, sw1):
        wid = jax.lax.axis_index("s") * info.num_cores + jax.lax.axis_index("c")
        base = wid * b_per_w
        pltpu.sync_copy(idx_hbm.at[pl.ds(base, b_per_w)], idx_v)

        @pl.loop(0, nsteps, step=2)
        def _(i):
            g0 = pltpu.async_copy(
                table_hbm.at[idx_v.at[pl.ds(i * ch, ch)]], buf0, sg0)
            g1 = pltpu.async_copy(
                table_hbm.at[idx_v.at[pl.ds((i + 1) * ch, ch)]], buf1, sg1)
            g0.wait()
            w0 = pltpu.async_copy(
                buf0, out_hbm.at[pl.ds(base + i * ch, ch)], sw0)
            g1.wait()
            w1 = pltpu.async_copy(
                buf1, out_hbm.at[pl.ds(base + (i + 1) * ch, ch)], sw1)
            w0.wait()
            w1.wait()

    return gather_kernel(table2, idx2)


def _conv_body(nb, mw):
    npos = mw - KSIZE + 1
    nquads = npos // 4

    def body(e_ref, wflat_ref, wpg_ref, cb1_ref, bpg_ref,
             out_ref, wpad_ref, uquad_ref):
        @pl.when(pl.program_id(0) == 0)
        def _build_tables():
            zpad = jnp.zeros((CPAD - ECHAR, EWORD), jnp.float32)
            pieces = []
            for k in range(KSIZE):
                pieces.append(wflat_ref[k * ECHAR:(k + 1) * ECHAR, :])
                pieces.append(zpad)
            wpad = jnp.concatenate(pieces, axis=0)
            wpad_ref[...] = wpad.astype(jnp.bfloat16)
            z64 = jnp.zeros((CPAD, EWORD), jnp.float32)
            cols = []
            for p in range(4):
                cols.append(jnp.concatenate(
                    [z64] * p + [wpad] + [z64] * (3 - p), axis=0))
            uquad_ref[...] = jnp.concatenate(cols, axis=1).astype(jnp.bfloat16)

        e = e_ref[...].astype(jnp.bfloat16)
        uquad = uquad_ref[...]
        m256 = None
        for q in range(nquads):
            base = 4 * q * CPAD
            a = jax.lax.dot_general(
                e[:, base:base + 8 * CPAD], uquad,
                (((1,), (0,)), ((), ())),
                preferred_element_type=jnp.float32)
            m256 = a if m256 is None else jnp.maximum(m256, a)
        m = jnp.maximum(
            jnp.maximum(m256[:, :EWORD], m256[:, EWORD:2 * EWORD]),
            jnp.maximum(m256[:, 2 * EWORD:3 * EWORD], m256[:, 3 * EWORD:]))
        for t in range(4 * nquads, npos):
            base = t * CPAD
            a = jax.lax.dot_general(
                e[:, base:base + KSIZE * CPAD], wpad_ref[...],
                (((1,), (0,)), ((), ())),
                preferred_element_type=jnp.float32)
            m = jnp.maximum(m, a)
        m = jnp.maximum(m + cb1_ref[...], 0.0)

        h = jax.lax.dot_general(
            m.astype(jnp.bfloat16), wpg_ref[...], (((1,), (0,)), ((), ())),
            preferred_element_type=jnp.float32) + bpg_ref[...]
        proj = jnp.maximum(h[:, :EWORD], 0.0)
        gate = jax.nn.sigmoid(h[:, EWORD:])
        out_ref[...] = gate * proj + (1.0 - gate) * m

    return body


def _conv_call(e, wflat, wpg, cb1, bpg, n, mw, ncols):
    nb = 2048 if n % 2048 == 0 else n
    grid = (n // nb,)
    return pl.pallas_call(
        _conv_body(nb, mw),
        grid=grid,
        in_specs=[
            pl.BlockSpec((nb, ncols), lambda i: (i, 0)),
            pl.BlockSpec((KSIZE * ECHAR, EWORD), lambda i: (0, 0)),
            pl.BlockSpec((EWORD, 2 * EWORD), lambda i: (0, 0)),
            pl.BlockSpec((1, EWORD), lambda i: (0, 0)),
            pl.BlockSpec((1, 2 * EWORD), lambda i: (0, 0)),
        ],
        out_specs=pl.BlockSpec((nb, EWORD), lambda i: (i, 0)),
        out_shape=jax.ShapeDtypeStruct((n, EWORD), jnp.float32),
        scratch_shapes=[
            pltpu.VMEM((KSIZE * CPAD, EWORD), jnp.bfloat16),
            pltpu.VMEM((8 * CPAD, 4 * EWORD), jnp.bfloat16),
        ],
    )(e, wflat, wpg, cb1, bpg)


def kernel(input, emb_table, conv_w, conv_b, W_proj, b_proj, W_gate, b_gate):
    sl, bs, mw = input.shape
    n = sl * bs
    npair = (mw + 1) // 2

    embp = jnp.pad(emb_table, ((0, 0), (0, CPAD - ECHAR)))
    table2 = jnp.concatenate(
        [jnp.repeat(embp, VOCAB, axis=0), jnp.tile(embp, (VOCAB, 1))],
        axis=1)
    wflat = conv_w.transpose(2, 1, 0).reshape(KSIZE * ECHAR, EWORD)
    wpg = jnp.concatenate([W_proj.T, W_gate.T], axis=1).astype(jnp.bfloat16)
    cb1 = conv_b[None, :]
    bpg = jnp.concatenate([b_proj, b_gate])[None, :]

    ii = input.reshape(n, mw)
    iipad = jnp.concatenate(
        [ii, jnp.zeros((n, 2 * npair - mw), jnp.int32)], axis=1)
    pairs = iipad.reshape(n, npair, 2)
    idx2 = (pairs[..., 0] * VOCAB + pairs[..., 1]).reshape(n * npair)

    eg = _sc_gather(table2, idx2)
    e = eg.reshape(n, npair * ROWW)

    out = _conv_call(e, wflat, wpg, cb1, bpg, n, mw, npair * ROWW)
    return out.reshape(sl, bs, EWORD)


# speedup vs baseline: 3.1835x; 2.7060x over previous
import jax
import jax.numpy as jnp
from jax.experimental import pallas as pl
from jax.experimental.pallas import tpu as pltpu

VOCAB = 96
VPAD = 128
ECHAR = 50
EWORD = 64
KSIZE = 5
NEG = -1.0e30


def _fused_kernel(nb, mw):
    npos = mw - KSIZE + 1
    nquads = (npos + 3) // 4
    nchars = 4 * (nquads - 1) + 8
    rem = npos - 4 * (nquads - 1)

    def body(idx_ref, emb_ref, wflat_ref, wpg_ref, cb1_ref, bpg_ref,
             out_ref, oh_ref, tquad_ref, tlast_ref):
        @pl.when(pl.program_id(0) == 0)
        def _build_tables():
            emb = emb_ref[...]
            z32 = jnp.zeros((VPAD - VOCAB, EWORD), jnp.float32)
            pieces = []
            for k in range(KSIZE):
                tk = emb @ wflat_ref[k * ECHAR:(k + 1) * ECHAR, :]
                pieces.append(tk)
                pieces.append(z32)
            tcat = jnp.concatenate(pieces, axis=0)
            z128 = jnp.zeros((VPAD, EWORD), jnp.float32)
            cols = []
            for q in range(4):
                col = jnp.concatenate([z128] * q + [tcat] + [z128] * (3 - q),
                                      axis=0)
                cols.append(col)
            tquad_ref[...] = jnp.concatenate(cols, axis=1).astype(jnp.bfloat16)
            realrows = (mw - 4 * (nquads - 1)) * VPAD
            lcols = cols[:rem]
            nmask = jnp.concatenate(
                [jnp.full((realrows, (4 - rem) * EWORD), NEG),
                 jnp.zeros((8 * VPAD - realrows, (4 - rem) * EWORD))], axis=0)
            tlast_ref[...] = jnp.concatenate(
                lcols + [nmask], axis=1).astype(jnp.bfloat16)
            oh_ref[:, mw * VPAD:] = jnp.zeros(
                (nb, (nchars - mw) * VPAD), jnp.bfloat16)

        idx = idx_ref[...]
        iota = jax.lax.broadcasted_iota(jnp.int32, (nb, VPAD), 1)
        iotab = iota.astype(jnp.bfloat16)
        one = jnp.ones((nb, VPAD), jnp.bfloat16)
        zero = jnp.zeros((nb, VPAD), jnp.bfloat16)
        for j in range(mw):
            oh_ref[:, j * VPAD:(j + 1) * VPAD] = jnp.where(
                idx[:, j][:, None] == iotab, one, zero)

        m256 = None
        for q in range(nquads):
            base = 4 * q * VPAD
            w = tquad_ref[...] if q < nquads - 1 else tlast_ref[...]
            a = jax.lax.dot_general(
                oh_ref[:, base:base + 8 * VPAD], w,
                (((1,), (0,)), ((), ())),
                preferred_element_type=jnp.float32)
            m256 = a if m256 is None else jnp.maximum(m256, a)
        m = jnp.maximum(
            jnp.maximum(m256[:, :EWORD], m256[:, EWORD:2 * EWORD]),
            jnp.maximum(m256[:, 2 * EWORD:3 * EWORD], m256[:, 3 * EWORD:]))
        m = jnp.maximum(m + cb1_ref[...], 0.0)

        h = jax.lax.dot_general(
            m.astype(jnp.bfloat16), wpg_ref[...], (((1,), (0,)), ((), ())),
            preferred_element_type=jnp.float32) + bpg_ref[...]
        proj = jnp.maximum(h[:, :EWORD], 0.0)
        gate = jax.nn.sigmoid(h[:, EWORD:])
        out_ref[...] = gate * proj + (1.0 - gate) * m

    return body


def kernel(input, emb_table, conv_w, conv_b, W_proj, b_proj, W_gate, b_gate):
    sl, bs, mw = input.shape
    n = sl * bs
    idx = input.reshape(n, mw).astype(jnp.bfloat16)

    wflat = conv_w.transpose(2, 1, 0).reshape(KSIZE * ECHAR, EWORD)
    wpg = jnp.concatenate([W_proj.T, W_gate.T], axis=1).astype(jnp.bfloat16)
    cb1 = conv_b[None, :]
    bpg = jnp.concatenate([b_proj, b_gate])[None, :]

    nb = 2048 if n % 2048 == 0 else n
    grid = (n // nb,)

    npos = mw - KSIZE + 1
    nquads = (npos + 3) // 4
    nchars = 4 * (nquads - 1) + 8

    out = pl.pallas_call(
        _fused_kernel(nb, mw),
        grid=grid,
        in_specs=[
            pl.BlockSpec((nb, mw), lambda i: (i, 0)),
            pl.BlockSpec((VOCAB, ECHAR), lambda i: (0, 0)),
            pl.BlockSpec((KSIZE * ECHAR, EWORD), lambda i: (0, 0)),
            pl.BlockSpec((EWORD, 2 * EWORD), lambda i: (0, 0)),
            pl.BlockSpec((1, EWORD), lambda i: (0, 0)),
            pl.BlockSpec((1, 2 * EWORD), lambda i: (0, 0)),
        ],
        out_specs=pl.BlockSpec((nb, EWORD), lambda i: (i, 0)),
        out_shape=jax.ShapeDtypeStruct((n, EWORD), jnp.float32),
        scratch_shapes=[
            pltpu.VMEM((nb, nchars * VPAD), jnp.bfloat16),
            pltpu.VMEM((8 * VPAD, 4 * EWORD), jnp.bfloat16),
            pltpu.VMEM((8 * VPAD, 4 * EWORD), jnp.bfloat16),
        ],
    )(idx, emb_table, wflat, wpg, cb1, bpg)
    return out.reshape(sl, bs, EWORD)


# speedup vs baseline: 3.2306x; 1.0148x over previous
import jax
import jax.numpy as jnp
from jax.experimental import pallas as pl
from jax.experimental.pallas import tpu as pltpu

VOCAB = 96
VPAD = 128
ECHAR = 50
EWORD = 64
KSIZE = 5


def _fused_kernel(nb, mw):
    npos = mw - KSIZE + 1
    nquads = (npos + 3) // 4
    nchars = 4 * (nquads - 1) + 8
    rem = npos - 4 * (nquads - 1)

    def body(idx_ref, emb_ref, wflat_ref, wpg_ref, cb1_ref, bpg_ref,
             out_ref, tquad_ref, tlast_ref):
        @pl.when(pl.program_id(0) == 0)
        def _build_tables():
            emb = emb_ref[...]
            z32 = jnp.zeros((VPAD - VOCAB, EWORD), jnp.float32)
            pieces = []
            for k in range(KSIZE):
                tk = emb @ wflat_ref[k * ECHAR:(k + 1) * ECHAR, :]
                pieces.append(tk)
                pieces.append(z32)
            tcat = jnp.concatenate(pieces, axis=0)
            z128 = jnp.zeros((VPAD, EWORD), jnp.float32)
            cols = []
            for q in range(4):
                col = jnp.concatenate([z128] * q + [tcat] + [z128] * (3 - q),
                                      axis=0)
                cols.append(col)
            tquad_ref[...] = jnp.concatenate(cols, axis=1).astype(jnp.bfloat16)
            realrows = (mw - 4 * (nquads - 1)) * VPAD
            nmask = jnp.concatenate(
                [jnp.full((realrows, (4 - rem) * EWORD), -1.0e30),
                 jnp.zeros((8 * VPAD - realrows, (4 - rem) * EWORD))], axis=0)
            tlast_ref[...] = jnp.concatenate(
                cols[:rem] + [nmask], axis=1).astype(jnp.bfloat16)

        idx = idx_ref[...]
        iota = jax.lax.broadcasted_iota(jnp.int32, (nb, VPAD), 1)
        iotab = iota.astype(jnp.bfloat16)
        one = jnp.ones((nb, VPAD), jnp.bfloat16)
        zero = jnp.zeros((nb, VPAD), jnp.bfloat16)
        oh = jnp.concatenate(
            [jnp.where(idx[:, j][:, None] == iotab, one, zero)
             for j in range(mw)] +
            [jnp.zeros((nb, (nchars - mw) * VPAD), jnp.bfloat16)],
            axis=1)

        tquad = tquad_ref[...]
        tlast = tlast_ref[...]
        m256 = None
        for q in range(nquads):
            base = 4 * q * VPAD
            a = jax.lax.dot_general(
                oh[:, base:base + 8 * VPAD],
                tquad if q < nquads - 1 else tlast,
                (((1,), (0,)), ((), ())),
                preferred_element_type=jnp.float32)
            m256 = a if m256 is None else jnp.maximum(m256, a)
        m = jnp.maximum(
            jnp.maximum(m256[:, :EWORD], m256[:, EWORD:2 * EWORD]),
            jnp.maximum(m256[:, 2 * EWORD:3 * EWORD], m256[:, 3 * EWORD:]))
        m = jnp.maximum(m + cb1_ref[...], 0.0)

        h = jax.lax.dot_general(
            m.astype(jnp.bfloat16), wpg_ref[...], (((1,), (0,)), ((), ())),
            preferred_element_type=jnp.float32) + bpg_ref[...]
        proj = jnp.maximum(h[:, :EWORD], 0.0)
        gate = jax.nn.sigmoid(h[:, EWORD:])
        out_ref[...] = gate * proj + (1.0 - gate) * m

    return body


def kernel(input, emb_table, conv_w, conv_b, W_proj, b_proj, W_gate, b_gate):
    sl, bs, mw = input.shape
    n = sl * bs
    idx = input.reshape(n, mw).astype(jnp.bfloat16)

    wflat = conv_w.transpose(2, 1, 0).reshape(KSIZE * ECHAR, EWORD)
    wpg = jnp.concatenate([W_proj.T, W_gate.T], axis=1).astype(jnp.bfloat16)
    cb1 = conv_b[None, :]
    bpg = jnp.concatenate([b_proj, b_gate])[None, :]

    nb = 2048 if n % 2048 == 0 else n
    grid = (n // nb,)

    out = pl.pallas_call(
        _fused_kernel(nb, mw),
        grid=grid,
        in_specs=[
            pl.BlockSpec((nb, mw), lambda i: (i, 0)),
            pl.BlockSpec((VOCAB, ECHAR), lambda i: (0, 0)),
            pl.BlockSpec((KSIZE * ECHAR, EWORD), lambda i: (0, 0)),
            pl.BlockSpec((EWORD, 2 * EWORD), lambda i: (0, 0)),
            pl.BlockSpec((1, EWORD), lambda i: (0, 0)),
            pl.BlockSpec((1, 2 * EWORD), lambda i: (0, 0)),
        ],
        out_specs=pl.BlockSpec((nb, EWORD), lambda i: (i, 0)),
        out_shape=jax.ShapeDtypeStruct((n, EWORD), jnp.float32),
        scratch_shapes=[
            pltpu.VMEM((8 * VPAD, 4 * EWORD), jnp.bfloat16),
            pltpu.VMEM((8 * VPAD, 4 * EWORD), jnp.bfloat16),
        ],
    )(idx, emb_table, wflat, wpg, cb1, bpg)
    return out.reshape(sl, bs, EWORD)
